# baseline (speedup 1.0000x reference)
import functools
import math

import jax
import jax.numpy as jnp
from jax import lax
from jax.experimental import pallas as pl
from jax.experimental.pallas import tpu as pltpu
from jax.experimental.pallas import tpu_sc as plsc

_INTERPRET = False

B = 16
NA = 3
NC = 80
NT = 4096
HWS = [4096, 1024, 256]
WS = [64, 32, 16]
NPOS = [196608, 49152, 12288]
MAPOFF = [0, 196608, 245760]
TOT = 258048
TOBJ_PAD = TOT + 2 * 3 * 61440
DUMMY = 258048
NE = 61440
BAL = [4.0, 1.0, 0.4]
ANCHORS_RAW = [[(10.0, 13.0), (16.0, 30.0), (33.0, 23.0)],
               [(30.0, 61.0), (62.0, 45.0), (59.0, 119.0)],
               [(116.0, 90.0), (156.0, 198.0), (373.0, 326.0)]]
STRIDES = [8.0, 16.0, 32.0]
ANCH = [[(a / s, b / s) for (a, b) in ANCHORS_RAW[i]] for i, s in enumerate(STRIDES)]
OFFS = [(0.0, 0.0), (0.5, 0.0), (0.0, 0.5), (-0.5, 0.0), (0.0, -0.5)]


def _prep_body(t_ref, idx_ref, f_ref):
    img = t_ref[0]
    cls = t_ref[1]
    x = t_ref[2]
    y = t_ref[3]
    w = t_ref[4]
    h = t_ref[5]
    b = img.astype(jnp.int32)
    tc = cls.astype(jnp.int32)
    for s in range(3):
        W = float(WS[s])
        HW = HWS[s]
        gx = x * W
        gy = y * W
        gw = w * W
        gh = h * W
        fx = gx - jnp.floor(gx)
        fy = gy - jnp.floor(gy)
        jj = (fx < 0.5) & (gx > 1.0)
        kk = (fy < 0.5) & (gy > 1.0)
        gxi = W - gx
        gyi = W - gy
        fxi = gxi - jnp.floor(gxi)
        fyi = gyi - jnp.floor(gyi)
        ll = (fxi < 0.5) & (gxi > 1.0)
        mm = (fyi < 0.5) & (gyi > 1.0)
        gates = [None, jj, kk, ll, mm]
        m0 = []
        for a in range(NA):
            aw, ah = ANCH[s][a]
            rw = gw * (1.0 / aw)
            rh = gh * (1.0 / ah)
            mw = jnp.maximum(rw, 1.0 / rw)
            mh = jnp.maximum(rh, 1.0 / rh)
            m0.append(jnp.maximum(mw, mh) < 4.0)
        for o in range(5):
            ox, oy = OFFS[o]
            gi = (gx - ox).astype(jnp.int32)
            gj = (gy - oy).astype(jnp.int32)
            gi = jnp.clip(gi, 0, WS[s] - 1)
            gj = jnp.clip(gj, 0, WS[s] - 1)
            tx = gx - gi.astype(jnp.float32)
            ty = gy - gj.astype(jnp.float32)
            pos = gj * WS[s] + gi
            for a in range(NA):
                ci = o * NA + a
                base = (b * 255 + 85 * a) * HW + pos
                for c in range(4):
                    idx_ref[s, c, ci] = base + c * HW
                idx_ref[s, 4, ci] = base + (5 + tc) * HW
                idx_ref[s, 5, ci] = (b * NA + a) * HW + pos + MAPOFF[s]
                if gates[o] is None:
                    mk = m0[a]
                else:
                    mk = gates[o] & m0[a]
                f_ref[s, 0, ci] = tx
                f_ref[s, 1, ci] = ty
                f_ref[s, 2, ci] = gw
                f_ref[s, 3, ci] = gh
                f_ref[s, 4, ci] = mk.astype(jnp.float32)


def _k_prep(tt):
    return pl.pallas_call(
        _prep_body,
        out_shape=(jax.ShapeDtypeStruct((3, 6, 15, 32, 128), jnp.int32),
                   jax.ShapeDtypeStruct((3, 5, 15, 32, 128), jnp.float32)),
        interpret=_INTERPRET,
    )(tt)


def _dense_body(x_ref, s_ref, d_ref, os_ref):
    rows = []
    for g in range(10):
        pr = 1.0 - x_ref[0, 5 + 8 * g]
        for k in range(1, 8):
            pr = pr * (1.0 - x_ref[0, 5 + 8 * g + k])
        rows.append(jnp.log(pr))
    acc = rows[0]
    for r in rows[1:]:
        acc = acc + r
    s_ref[0] = acc
    po = x_ref[0, 4]
    lo1 = jnp.log(1.0 - po)
    lo0 = jnp.log(po)
    d_ref[0] = lo1 - lo0
    os_ref[0, 0] = jnp.sum(lo1, axis=0)


def _k_dense(pred, s):
    hw = HWS[s]
    sub = hw // 128
    p = pred.reshape(B, 255, sub, 128)
    grid = (B * NA,)
    return pl.pallas_call(
        _dense_body,
        grid=grid,
        in_specs=[pl.BlockSpec((1, 85, sub, 128), lambda i: (i // 3, i % 3, 0, 0))],
        out_specs=(pl.BlockSpec((1, sub, 128), lambda i: (i, 0, 0)),
                   pl.BlockSpec((1, sub, 128), lambda i: (i, 0, 0)),
                   pl.BlockSpec((1, 1, 128), lambda i: (i, 0, 0))),
        out_shape=(jax.ShapeDtypeStruct((B * NA, sub, 128), jnp.float32),
                   jax.ShapeDtypeStruct((B * NA, sub, 128), jnp.float32),
                   jax.ShapeDtypeStruct((B * NA, 1, 128), jnp.float32)),
        interpret=_INTERPRET,
    )(p)


def _atan_pos(z):
    inv = z > 1.0
    zz = jnp.where(inv, 1.0 / z, z)
    x2 = zz * zz
    p = -0.0117212
    p = p * x2 + 0.0529126
    p = p * x2 - 0.1169414
    p = p * x2 + 0.1939339
    p = p * x2 - 0.3326221
    p = p * x2 + 0.9999791
    at = p * zz
    return jnp.where(inv, (math.pi / 2.0) - at, at)


def _entry_body(g_ref, f_ref, im_ref, val_ref, sidx_ref, sums_ref):
    for s in range(3):
        acc_sl1 = jnp.zeros((32, 128), jnp.float32)
        acc_bce = jnp.zeros((32, 128), jnp.float32)
        acc_cnt = jnp.zeros((32, 128), jnp.float32)
        for ci in range(15):
            a = ci % NA
            aw, ah = ANCH[s][a]
            p0 = g_ref[s, 0, ci]
            p1 = g_ref[s, 1, ci]
            p2 = g_ref[s, 2, ci]
            p3 = g_ref[s, 3, ci]
            pct = g_ref[s, 4, ci]
            sv = g_ref[s, 5, ci]
            tx = f_ref[s, 0, ci]
            ty = f_ref[s, 1, ci]
            tw = f_ref[s, 2, ci]
            th = f_ref[s, 3, ci]
            mk = f_ref[s, 4, ci]
            px = p0 * 2.0 - 0.5
            py = p1 * 2.0 - 0.5
            pw = (p2 * 2.0) ** 2 * aw
            ph = (p3 * 2.0) ** 2 * ah
            sl1 = jnp.zeros((32, 128), jnp.float32)
            for pv, tv in ((px, tx), (py, ty), (pw, tw), (ph, th)):
                d = jnp.abs(pv - tv)
                sl1 = sl1 + jnp.where(d < 1.0, 0.5 * d * d, d - 0.5)
            acc_sl1 = acc_sl1 + mk * sl1
            bce = -jnp.log(pct) + jnp.log(1.0 - pct) - sv
            acc_bce = acc_bce + mk * bce
            acc_cnt = acc_cnt + mk
            b1x1 = px - pw * 0.5
            b1x2 = px + pw * 0.5
            b1y1 = py - ph * 0.5
            b1y2 = py + ph * 0.5
            b2x1 = tx - tw * 0.5
            b2x2 = tx + tw * 0.5
            b2y1 = ty - th * 0.5
            b2y2 = ty + th * 0.5
            iw = jnp.maximum(jnp.minimum(b1x2, b2x2) - jnp.maximum(b1x1, b2x1), 0.0)
            ih = jnp.maximum(jnp.minimum(b1y2, b2y2) - jnp.maximum(b1y1, b2y1), 0.0)
            inter = iw * ih
            union = pw * ph + 1e-16 + tw * th - inter
            iou = inter / union
            cw = jnp.maximum(b1x2, b2x2) - jnp.minimum(b1x1, b2x1)
            ch = jnp.maximum(b1y2, b2y2) - jnp.minimum(b1y1, b2y1)
            c2 = cw * cw + ch * ch + 1e-16
            rho2 = ((b2x1 + b2x2 - b1x1 - b1x2) ** 2
                    + (b2y1 + b2y2 - b1y1 - b1y2) ** 2) * 0.25
            v = (4.0 / (math.pi ** 2)) * (_atan_pos(tw / th) - _atan_pos(pw / ph)) ** 2
            alpha = v / (1.0 - iou + v + 1e-16)
            ciou = iou - (rho2 / c2 + v * alpha)
            val_ref[s, ci] = 0.5 + 0.5 * jnp.maximum(ciou, 0.0)
            eg = ((s * 15 + ci) * 32 * 128
                  + lax.broadcasted_iota(jnp.int32, (32, 128), 0) * 128
                  + lax.broadcasted_iota(jnp.int32, (32, 128), 1))
            im = im_ref[s, ci]
            ok = mk > 0.5
            half = TOT // 2
            sidx_ref[0, s, ci] = jnp.where(ok & (im < half), im, TOT + eg)
            sidx_ref[1, s, ci] = jnp.where(ok & (im >= half), im,
                                           TOT + 3 * NE + eg)
        sums_ref[s, 0] = jnp.sum(acc_sl1, axis=0)
        sums_ref[s, 1] = jnp.sum(acc_bce, axis=0)
        sums_ref[s, 2] = jnp.sum(acc_cnt, axis=0)
        for r in range(3, 8):
            sums_ref[s, r] = jnp.zeros((128,), jnp.float32)


def _k_entry(g, ft, im):
    return pl.pallas_call(
        _entry_body,
        out_shape=(jax.ShapeDtypeStruct((3, 15, 32, 128), jnp.float32),
                   jax.ShapeDtypeStruct((2, 3, 15, 32, 128), jnp.int32),
                   jax.ShapeDtypeStruct((3, 8, 128), jnp.float32)),
        interpret=_INTERPRET,
    )(g, ft, im)


def _fin_body(tobj_ref, d0_ref, d1_ref, d2_ref, o0_ref, o1_ref, o2_ref,
              sums_ref, out_ref):
    r0 = 1536
    r1 = 1920
    st = [jnp.sum(tobj_ref[0:r0] * d0_ref[...]),
          jnp.sum(tobj_ref[r0:r1] * d1_ref[...]),
          jnp.sum(tobj_ref[r1:2016] * d2_ref[...])]
    osum = [jnp.sum(o0_ref[...]), jnp.sum(o1_ref[...]), jnp.sum(o2_ref[...])]
    lobj = jnp.float32(0.0)
    lbox = jnp.float32(0.0)
    lcls = jnp.float32(0.0)
    for s in range(3):
        lobj = lobj + BAL[s] * (-osum[s] + st[s]) / float(NPOS[s])
        sl1 = jnp.sum(sums_ref[s, 0])
        bce = jnp.sum(sums_ref[s, 1])
        cnt = jnp.sum(sums_ref[s, 2])
        den = jnp.maximum(cnt, 1.0)
        lbox = lbox + sl1 / (den * 4.0)
        lcls = lcls + bce / (den * float(NC))
    lbox = lbox * 0.05
    lobj = lobj * 1.4
    lcls = lcls * 0.5
    loss = (lbox + lobj + lcls) * float(B)
    out_ref[0] = jnp.full((128,), loss, jnp.float32)
    out_ref[1] = jnp.full((128,), lbox, jnp.float32)
    out_ref[2] = jnp.full((128,), lobj, jnp.float32)
    out_ref[3] = jnp.full((128,), lcls, jnp.float32)


def _k_fin(tobj2d, d0, d1, d2, o0, o1, o2, sums):
    return pl.pallas_call(
        _fin_body,
        out_shape=jax.ShapeDtypeStruct((4, 128), jnp.float32),
        interpret=_INTERPRET,
    )(tobj2d, d0, d1, d2, o0, o1, o2, sums)


_HALF = TOT // 2
_ZPT = _HALF // 16
_EPT = (3 * NE) // 16
_NCH = _EPT // 128


def _scatter_body(sidx_hbm, val_hbm, zer_hbm, tobj_hbm, idx_v, val_v, zer_v, sem):
    cid = lax.axis_index("c")
    sid = lax.axis_index("s")
    pltpu.sync_copy(zer_hbm, zer_v)
    pltpu.sync_copy(zer_v, tobj_hbm.at[pl.ds(cid * _HALF + sid * _ZPT, _ZPT)])
    plsc.subcore_barrier()
    pltpu.sync_copy(sidx_hbm.at[cid, sid], idx_v)
    pltpu.sync_copy(val_hbm.at[sid], val_v)
    pltpu.async_copy(val_v, tobj_hbm.at[idx_v], sem).wait()


def _sc_scatter(sidx2, val):
    f = pl.kernel(
        _scatter_body,
        out_type=jax.ShapeDtypeStruct((TOBJ_PAD,), jnp.float32),
        mesh=plsc.VectorSubcoreMesh(core_axis_name="c", subcore_axis_name="s"),
        scratch_types=[
            pltpu.VMEM((_EPT,), jnp.int32),
            pltpu.VMEM((_EPT,), jnp.float32),
            pltpu.VMEM((_ZPT,), jnp.float32),
            pltpu.SemaphoreType.DMA,
        ],
    )
    zer = jnp.zeros((_ZPT,), jnp.float32)
    return f(sidx2.reshape(2, 16, _EPT), val.reshape(16, _EPT), zer)


def kernel(pred0, pred1, pred2, targets):
    preds = [pred0, pred1, pred2]
    tt = targets.T.reshape(6, 32, 128)
    idx, ft = _k_prep(tt)
    dense = [_k_dense(preds[s], s) for s in range(3)]
    sflat = jnp.concatenate([dense[s][0].reshape(-1) for s in range(3)])
    gs = []
    for s in range(3):
        pf = preds[s].reshape(-1)
        gb = pf[idx[s, 0:5].reshape(5, -1)]
        gm = sflat[idx[s, 5].reshape(1, -1)]
        gs.append(jnp.concatenate([gb, gm], axis=0))
    g = jnp.stack(gs).reshape(3, 6, 15, 32, 128)
    val, sidx, sums = _k_entry(g, ft, idx[:, 5])
    tobj = _sc_scatter(sidx, val)
    tobj2d = tobj[:TOT].reshape(2016, 128)
    out = _k_fin(tobj2d,
                 dense[0][1].reshape(1536, 128),
                 dense[1][1].reshape(384, 128),
                 dense[2][1].reshape(96, 128),
                 dense[0][2].reshape(48, 128),
                 dense[1][2].reshape(48, 128),
                 dense[2][2].reshape(48, 128),
                 sums)
    return out[0, :1], out[1:4, 0]


# speedup vs baseline: 2.4046x; 2.4046x over previous
import functools
import math

import jax
import jax.numpy as jnp
from jax import lax
from jax.experimental import pallas as pl
from jax.experimental.pallas import tpu as pltpu
from jax.experimental.pallas import tpu_sc as plsc

_INTERPRET = False

B = 16
NA = 3
NC = 80
NT = 4096
HWS = [4096, 1024, 256]
WS = [64, 32, 16]
NPOS = [196608, 49152, 12288]
MAPOFF = [0, 196608, 245760]
TOT = 258048
TOBJ_PAD = TOT + 2 * 3 * 61440
DUMMY = 258048
NE = 61440
BAL = [4.0, 1.0, 0.4]
ANCHORS_RAW = [[(10.0, 13.0), (16.0, 30.0), (33.0, 23.0)],
               [(30.0, 61.0), (62.0, 45.0), (59.0, 119.0)],
               [(116.0, 90.0), (156.0, 198.0), (373.0, 326.0)]]
STRIDES = [8.0, 16.0, 32.0]
ANCH = [[(a / s, b / s) for (a, b) in ANCHORS_RAW[i]] for i, s in enumerate(STRIDES)]
OFFS = [(0.0, 0.0), (0.5, 0.0), (0.0, 0.5), (-0.5, 0.0), (0.0, -0.5)]


def _prep_body(t_ref, idx_ref, f_ref):
    img = t_ref[0]
    cls = t_ref[1]
    x = t_ref[2]
    y = t_ref[3]
    w = t_ref[4]
    h = t_ref[5]
    b = img.astype(jnp.int32)
    tc = cls.astype(jnp.int32)
    for s in range(3):
        W = float(WS[s])
        HW = HWS[s]
        gx = x * W
        gy = y * W
        gw = w * W
        gh = h * W
        fx = gx - jnp.floor(gx)
        fy = gy - jnp.floor(gy)
        jj = (fx < 0.5) & (gx > 1.0)
        kk = (fy < 0.5) & (gy > 1.0)
        gxi = W - gx
        gyi = W - gy
        fxi = gxi - jnp.floor(gxi)
        fyi = gyi - jnp.floor(gyi)
        ll = (fxi < 0.5) & (gxi > 1.0)
        mm = (fyi < 0.5) & (gyi > 1.0)
        gates = [None, jj, kk, ll, mm]
        m0 = []
        for a in range(NA):
            aw, ah = ANCH[s][a]
            rw = gw * (1.0 / aw)
            rh = gh * (1.0 / ah)
            mw = jnp.maximum(rw, 1.0 / rw)
            mh = jnp.maximum(rh, 1.0 / rh)
            m0.append(jnp.maximum(mw, mh) < 4.0)
        for o in range(5):
            ox, oy = OFFS[o]
            gi = (gx - ox).astype(jnp.int32)
            gj = (gy - oy).astype(jnp.int32)
            gi = jnp.clip(gi, 0, WS[s] - 1)
            gj = jnp.clip(gj, 0, WS[s] - 1)
            tx = gx - gi.astype(jnp.float32)
            ty = gy - gj.astype(jnp.float32)
            pos = gj * WS[s] + gi
            for a in range(NA):
                ci = o * NA + a
                base = (b * 255 + 85 * a) * HW + pos
                for c in range(4):
                    idx_ref[s, c, ci] = base + c * HW
                idx_ref[s, 4, ci] = base + (5 + tc) * HW
                idx_ref[s, 5, ci] = (b * NA + a) * HW + pos + MAPOFF[s]
                if gates[o] is None:
                    mk = m0[a]
                else:
                    mk = gates[o] & m0[a]
                f_ref[s, 0, ci] = tx
                f_ref[s, 1, ci] = ty
                f_ref[s, 2, ci] = gw
                f_ref[s, 3, ci] = gh
                f_ref[s, 4, ci] = mk.astype(jnp.float32)


def _k_prep(tt):
    return pl.pallas_call(
        _prep_body,
        out_shape=(jax.ShapeDtypeStruct((3, 6, 15, 32, 128), jnp.int32),
                   jax.ShapeDtypeStruct((3, 5, 15, 32, 128), jnp.float32)),
        interpret=_INTERPRET,
    )(tt)


def _dense_body(x_ref, s_ref, d_ref, os_ref):
    rows = []
    for g in range(10):
        pr = 1.0 - x_ref[0, 5 + 8 * g]
        for k in range(1, 8):
            pr = pr * (1.0 - x_ref[0, 5 + 8 * g + k])
        rows.append(jnp.log(pr))
    acc = rows[0]
    for r in rows[1:]:
        acc = acc + r
    s_ref[0] = acc
    po = x_ref[0, 4]
    lo1 = jnp.log(1.0 - po)
    lo0 = jnp.log(po)
    d_ref[0] = lo1 - lo0
    os_ref[0, 0] = jnp.sum(lo1, axis=0)


def _k_dense(pred, s):
    hw = HWS[s]
    sub = hw // 128
    p = pred.reshape(B, 255, sub, 128)
    grid = (B * NA,)
    return pl.pallas_call(
        _dense_body,
        grid=grid,
        in_specs=[pl.BlockSpec((1, 85, sub, 128), lambda i: (i // 3, i % 3, 0, 0))],
        out_specs=(pl.BlockSpec((1, sub, 128), lambda i: (i, 0, 0)),
                   pl.BlockSpec((1, sub, 128), lambda i: (i, 0, 0)),
                   pl.BlockSpec((1, 1, 128), lambda i: (i, 0, 0))),
        out_shape=(jax.ShapeDtypeStruct((B * NA, sub, 128), jnp.float32),
                   jax.ShapeDtypeStruct((B * NA, sub, 128), jnp.float32),
                   jax.ShapeDtypeStruct((B * NA, 1, 128), jnp.float32)),
        interpret=_INTERPRET,
    )(p)


def _atan_pos(z):
    inv = z > 1.0
    zz = jnp.where(inv, 1.0 / z, z)
    x2 = zz * zz
    p = -0.0117212
    p = p * x2 + 0.0529126
    p = p * x2 - 0.1169414
    p = p * x2 + 0.1939339
    p = p * x2 - 0.3326221
    p = p * x2 + 0.9999791
    at = p * zz
    return jnp.where(inv, (math.pi / 2.0) - at, at)


def _entry_body(g_ref, f_ref, im_ref, val_ref, sidx_ref, sums_ref):
    for s in range(3):
        acc_sl1 = jnp.zeros((32, 128), jnp.float32)
        acc_bce = jnp.zeros((32, 128), jnp.float32)
        acc_cnt = jnp.zeros((32, 128), jnp.float32)
        for ci in range(15):
            a = ci % NA
            aw, ah = ANCH[s][a]
            p0 = g_ref[s, 0, ci]
            p1 = g_ref[s, 1, ci]
            p2 = g_ref[s, 2, ci]
            p3 = g_ref[s, 3, ci]
            pct = g_ref[s, 4, ci]
            sv = g_ref[s, 5, ci]
            tx = f_ref[s, 0, ci]
            ty = f_ref[s, 1, ci]
            tw = f_ref[s, 2, ci]
            th = f_ref[s, 3, ci]
            mk = f_ref[s, 4, ci]
            px = p0 * 2.0 - 0.5
            py = p1 * 2.0 - 0.5
            pw = (p2 * 2.0) ** 2 * aw
            ph = (p3 * 2.0) ** 2 * ah
            sl1 = jnp.zeros((32, 128), jnp.float32)
            for pv, tv in ((px, tx), (py, ty), (pw, tw), (ph, th)):
                d = jnp.abs(pv - tv)
                sl1 = sl1 + jnp.where(d < 1.0, 0.5 * d * d, d - 0.5)
            acc_sl1 = acc_sl1 + mk * sl1
            bce = -jnp.log(pct) + jnp.log(1.0 - pct) - sv
            acc_bce = acc_bce + mk * bce
            acc_cnt = acc_cnt + mk
            b1x1 = px - pw * 0.5
            b1x2 = px + pw * 0.5
            b1y1 = py - ph * 0.5
            b1y2 = py + ph * 0.5
            b2x1 = tx - tw * 0.5
            b2x2 = tx + tw * 0.5
            b2y1 = ty - th * 0.5
            b2y2 = ty + th * 0.5
            iw = jnp.maximum(jnp.minimum(b1x2, b2x2) - jnp.maximum(b1x1, b2x1), 0.0)
            ih = jnp.maximum(jnp.minimum(b1y2, b2y2) - jnp.maximum(b1y1, b2y1), 0.0)
            inter = iw * ih
            union = pw * ph + 1e-16 + tw * th - inter
            iou = inter / union
            cw = jnp.maximum(b1x2, b2x2) - jnp.minimum(b1x1, b2x1)
            ch = jnp.maximum(b1y2, b2y2) - jnp.minimum(b1y1, b2y1)
            c2 = cw * cw + ch * ch + 1e-16
            rho2 = ((b2x1 + b2x2 - b1x1 - b1x2) ** 2
                    + (b2y1 + b2y2 - b1y1 - b1y2) ** 2) * 0.25
            v = (4.0 / (math.pi ** 2)) * (_atan_pos(tw / th) - _atan_pos(pw / ph)) ** 2
            alpha = v / (1.0 - iou + v + 1e-16)
            ciou = iou - (rho2 / c2 + v * alpha)
            val_ref[s, ci] = 0.5 + 0.5 * jnp.maximum(ciou, 0.0)
            eg = ((s * 15 + ci) * 32 * 128
                  + lax.broadcasted_iota(jnp.int32, (32, 128), 0) * 128
                  + lax.broadcasted_iota(jnp.int32, (32, 128), 1))
            egl = jnp.where(eg < 3 * NE // 2, eg, eg - 3 * NE // 2)
            im = im_ref[s, ci]
            sidx_ref[s, ci] = jnp.where(mk > 0.5, im, TOT + egl)
        sums_ref[s, 0] = jnp.sum(acc_sl1, axis=0)
        sums_ref[s, 1] = jnp.sum(acc_bce, axis=0)
        sums_ref[s, 2] = jnp.sum(acc_cnt, axis=0)
        for r in range(3, 8):
            sums_ref[s, r] = jnp.zeros((128,), jnp.float32)


def _k_entry(g, ft, im):
    return pl.pallas_call(
        _entry_body,
        out_shape=(jax.ShapeDtypeStruct((3, 15, 32, 128), jnp.float32),
                   jax.ShapeDtypeStruct((3, 15, 32, 128), jnp.int32),
                   jax.ShapeDtypeStruct((3, 8, 128), jnp.float32)),
        interpret=_INTERPRET,
    )(g, ft, im)


def _fin_body(tobj_ref, d0_ref, d1_ref, d2_ref, o0_ref, o1_ref, o2_ref,
              sums_ref, out_ref):
    r0 = 1536
    r1 = 1920
    tob = jnp.maximum(tobj_ref[0], tobj_ref[1])
    st = [jnp.sum(tob[0:r0] * d0_ref[...]),
          jnp.sum(tob[r0:r1] * d1_ref[...]),
          jnp.sum(tob[r1:2016] * d2_ref[...])]
    osum = [jnp.sum(o0_ref[...]), jnp.sum(o1_ref[...]), jnp.sum(o2_ref[...])]
    lobj = jnp.float32(0.0)
    lbox = jnp.float32(0.0)
    lcls = jnp.float32(0.0)
    for s in range(3):
        lobj = lobj + BAL[s] * (-osum[s] + st[s]) / float(NPOS[s])
        sl1 = jnp.sum(sums_ref[s, 0])
        bce = jnp.sum(sums_ref[s, 1])
        cnt = jnp.sum(sums_ref[s, 2])
        den = jnp.maximum(cnt, 1.0)
        lbox = lbox + sl1 / (den * 4.0)
        lcls = lcls + bce / (den * float(NC))
    lbox = lbox * 0.05
    lobj = lobj * 1.4
    lcls = lcls * 0.5
    loss = (lbox + lobj + lcls) * float(B)
    out_ref[0] = jnp.full((128,), loss, jnp.float32)
    out_ref[1] = jnp.full((128,), lbox, jnp.float32)
    out_ref[2] = jnp.full((128,), lobj, jnp.float32)
    out_ref[3] = jnp.full((128,), lcls, jnp.float32)


def _k_fin(tobj2d, d0, d1, d2, o0, o1, o2, sums):
    return pl.pallas_call(
        _fin_body,
        out_shape=jax.ShapeDtypeStruct((4, 128), jnp.float32),
        interpret=_INTERPRET,
    )(tobj2d, d0, d1, d2, o0, o1, o2, sums)


_EPW = (3 * NE) // 32
_SPM = TOT + (3 * NE) // 2
_MPT = TOT // 16


def _scatter_body(sidx_hbm, val_hbm, zer_hbm, out_hbm, idx_v, val_v, spm, sem):
    cid = lax.axis_index("c")
    sid = lax.axis_index("s")
    w = cid * 16 + sid
    pltpu.sync_copy(zer_hbm, spm.at[pl.ds(sid * _MPT, _MPT)])
    plsc.subcore_barrier()
    pltpu.sync_copy(sidx_hbm.at[w], idx_v)
    pltpu.sync_copy(val_hbm.at[w], val_v)
    pltpu.async_copy(val_v, spm.at[idx_v], sem).wait()
    plsc.subcore_barrier()
    pltpu.sync_copy(spm.at[pl.ds(sid * _MPT, _MPT)],
                    out_hbm.at[cid, pl.ds(sid * _MPT, _MPT)])


def _sc_scatter(sidx, val):
    f = pl.kernel(
        _scatter_body,
        out_type=jax.ShapeDtypeStruct((2, TOT), jnp.float32),
        mesh=plsc.VectorSubcoreMesh(core_axis_name="c", subcore_axis_name="s"),
        scratch_types=[
            pltpu.VMEM((_EPW,), jnp.int32),
            pltpu.VMEM((_EPW,), jnp.float32),
            pltpu.VMEM_SHARED((_SPM,), jnp.float32),
            pltpu.SemaphoreType.DMA,
        ],
    )
    zer = jnp.zeros((_MPT,), jnp.float32)
    return f(sidx.reshape(32, _EPW), val.reshape(32, _EPW), zer)


def kernel(pred0, pred1, pred2, targets):
    preds = [pred0, pred1, pred2]
    tt = targets.T.reshape(6, 32, 128)
    idx, ft = _k_prep(tt)
    dense = [_k_dense(preds[s], s) for s in range(3)]
    sflat = jnp.concatenate([dense[s][0].reshape(-1) for s in range(3)])
    gs = []
    for s in range(3):
        pf = preds[s].reshape(-1)
        gb = pf[idx[s, 0:5].reshape(5, -1)]
        gm = sflat[idx[s, 5].reshape(1, -1)]
        gs.append(jnp.concatenate([gb, gm], axis=0))
    g = jnp.stack(gs).reshape(3, 6, 15, 32, 128)
    val, sidx, sums = _k_entry(g, ft, idx[:, 5])
    tobj2d = _sc_scatter(sidx, val).reshape(2, 2016, 128)
    out = _k_fin(tobj2d,
                 dense[0][1].reshape(1536, 128),
                 dense[1][1].reshape(384, 128),
                 dense[2][1].reshape(96, 128),
                 dense[0][2].reshape(48, 128),
                 dense[1][2].reshape(48, 128),
                 dense[2][2].reshape(48, 128),
                 sums)
    return out[0, :1], out[1:4, 0]


# speedup vs baseline: 2.6168x; 1.0882x over previous
import functools
import math

import jax
import jax.numpy as jnp
from jax import lax
from jax.experimental import pallas as pl
from jax.experimental.pallas import tpu as pltpu
from jax.experimental.pallas import tpu_sc as plsc

_INTERPRET = False

B = 16
NA = 3
NC = 80
NT = 4096
HWS = [4096, 1024, 256]
WS = [64, 32, 16]
NPOS = [196608, 49152, 12288]
MAPOFF = [0, 196608, 245760]
TOT = 258048
TOBJ_PAD = TOT + 2 * 3 * 61440
DUMMY = 258048
NE = 61440
BAL = [4.0, 1.0, 0.4]
ANCHORS_RAW = [[(10.0, 13.0), (16.0, 30.0), (33.0, 23.0)],
               [(30.0, 61.0), (62.0, 45.0), (59.0, 119.0)],
               [(116.0, 90.0), (156.0, 198.0), (373.0, 326.0)]]
STRIDES = [8.0, 16.0, 32.0]
ANCH = [[(a / s, b / s) for (a, b) in ANCHORS_RAW[i]] for i, s in enumerate(STRIDES)]
OFFS = [(0.0, 0.0), (0.5, 0.0), (0.0, 0.5), (-0.5, 0.0), (0.0, -0.5)]


def _prep_body(t_ref, idx_ref, f_ref):
    img = t_ref[0]
    cls = t_ref[1]
    x = t_ref[2]
    y = t_ref[3]
    w = t_ref[4]
    h = t_ref[5]
    b = img.astype(jnp.int32)
    tc = cls.astype(jnp.int32)
    for s in range(3):
        W = float(WS[s])
        HW = HWS[s]
        gx = x * W
        gy = y * W
        gw = w * W
        gh = h * W
        fx = gx - jnp.floor(gx)
        fy = gy - jnp.floor(gy)
        jj = (fx < 0.5) & (gx > 1.0)
        kk = (fy < 0.5) & (gy > 1.0)
        gxi = W - gx
        gyi = W - gy
        fxi = gxi - jnp.floor(gxi)
        fyi = gyi - jnp.floor(gyi)
        ll = (fxi < 0.5) & (gxi > 1.0)
        mm = (fyi < 0.5) & (gyi > 1.0)
        gates = [None, jj, kk, ll, mm]
        m0 = []
        for a in range(NA):
            aw, ah = ANCH[s][a]
            rw = gw * (1.0 / aw)
            rh = gh * (1.0 / ah)
            mw = jnp.maximum(rw, 1.0 / rw)
            mh = jnp.maximum(rh, 1.0 / rh)
            m0.append(jnp.maximum(mw, mh) < 4.0)
        for o in range(5):
            ox, oy = OFFS[o]
            gi = (gx - ox).astype(jnp.int32)
            gj = (gy - oy).astype(jnp.int32)
            gi = jnp.clip(gi, 0, WS[s] - 1)
            gj = jnp.clip(gj, 0, WS[s] - 1)
            tx = gx - gi.astype(jnp.float32)
            ty = gy - gj.astype(jnp.float32)
            pos = gj * WS[s] + gi
            for a in range(NA):
                ci = o * NA + a
                base = (b * 255 + 85 * a) * HW + pos
                for c in range(4):
                    idx_ref[s, c, ci] = base + c * HW
                idx_ref[s, 4, ci] = base + (5 + tc) * HW
                idx_ref[s, 5, ci] = (b * NA + a) * HW + pos
                if gates[o] is None:
                    mk = m0[a]
                else:
                    mk = gates[o] & m0[a]
                f_ref[s, 0, ci] = tx
                f_ref[s, 1, ci] = ty
                f_ref[s, 2, ci] = gw
                f_ref[s, 3, ci] = gh
                f_ref[s, 4, ci] = mk.astype(jnp.float32)


def _k_prep(tt):
    return pl.pallas_call(
        _prep_body,
        out_shape=(jax.ShapeDtypeStruct((3, 6, 15, 32, 128), jnp.int32),
                   jax.ShapeDtypeStruct((3, 5, 15, 32, 128), jnp.float32)),
        interpret=_INTERPRET,
    )(tt)


def _dense_body(x_ref, s_ref, d_ref, os_ref):
    rows = []
    for g in range(10):
        pr = 1.0 - x_ref[0, 5 + 8 * g]
        for k in range(1, 8):
            pr = pr * (1.0 - x_ref[0, 5 + 8 * g + k])
        rows.append(jnp.log(pr))
    acc = rows[0]
    for r in rows[1:]:
        acc = acc + r
    s_ref[0] = acc
    po = x_ref[0, 4]
    lo1 = jnp.log(1.0 - po)
    lo0 = jnp.log(po)
    d_ref[0] = lo1 - lo0
    os_ref[0, 0] = jnp.sum(lo1, axis=0)


def _k_dense(p, s):
    hw = HWS[s]
    sub = hw // 128
    grid = (B * NA,)
    return pl.pallas_call(
        _dense_body,
        grid=grid,
        in_specs=[pl.BlockSpec((1, 85, sub, 128), lambda i: (i // 3, i % 3, 0, 0))],
        out_specs=(pl.BlockSpec((1, sub, 128), lambda i: (i, 0, 0)),
                   pl.BlockSpec((1, sub, 128), lambda i: (i, 0, 0)),
                   pl.BlockSpec((1, 1, 128), lambda i: (i, 0, 0))),
        out_shape=(jax.ShapeDtypeStruct((B * NA, sub, 128), jnp.float32),
                   jax.ShapeDtypeStruct((B * NA, sub, 128), jnp.float32),
                   jax.ShapeDtypeStruct((B * NA, 1, 128), jnp.float32)),
        interpret=_INTERPRET,
    )(p)


def _atan_pos(z):
    inv = z > 1.0
    zz = jnp.where(inv, 1.0 / z, z)
    x2 = zz * zz
    p = -0.0117212
    p = p * x2 + 0.0529126
    p = p * x2 - 0.1169414
    p = p * x2 + 0.1939339
    p = p * x2 - 0.3326221
    p = p * x2 + 0.9999791
    at = p * zz
    return jnp.where(inv, (math.pi / 2.0) - at, at)


def _entry_body(g_ref, f_ref, im_ref, val_ref, sidx_ref, sums_ref):
    for s in range(3):
        acc_sl1 = jnp.zeros((32, 128), jnp.float32)
        acc_bce = jnp.zeros((32, 128), jnp.float32)
        acc_cnt = jnp.zeros((32, 128), jnp.float32)
        for ci in range(15):
            a = ci % NA
            aw, ah = ANCH[s][a]
            p0 = g_ref[s, 0, ci]
            p1 = g_ref[s, 1, ci]
            p2 = g_ref[s, 2, ci]
            p3 = g_ref[s, 3, ci]
            pct = g_ref[s, 4, ci]
            sv = g_ref[s, 5, ci]
            tx = f_ref[s, 0, ci]
            ty = f_ref[s, 1, ci]
            tw = f_ref[s, 2, ci]
            th = f_ref[s, 3, ci]
            mk = f_ref[s, 4, ci]
            px = p0 * 2.0 - 0.5
            py = p1 * 2.0 - 0.5
            pw = (p2 * 2.0) ** 2 * aw
            ph = (p3 * 2.0) ** 2 * ah
            sl1 = jnp.zeros((32, 128), jnp.float32)
            for pv, tv in ((px, tx), (py, ty), (pw, tw), (ph, th)):
                d = jnp.abs(pv - tv)
                sl1 = sl1 + jnp.where(d < 1.0, 0.5 * d * d, d - 0.5)
            acc_sl1 = acc_sl1 + mk * sl1
            bce = -jnp.log(pct) + jnp.log(1.0 - pct) - sv
            acc_bce = acc_bce + mk * bce
            acc_cnt = acc_cnt + mk
            b1x1 = px - pw * 0.5
            b1x2 = px + pw * 0.5
            b1y1 = py - ph * 0.5
            b1y2 = py + ph * 0.5
            b2x1 = tx - tw * 0.5
            b2x2 = tx + tw * 0.5
            b2y1 = ty - th * 0.5
            b2y2 = ty + th * 0.5
            iw = jnp.maximum(jnp.minimum(b1x2, b2x2) - jnp.maximum(b1x1, b2x1), 0.0)
            ih = jnp.maximum(jnp.minimum(b1y2, b2y2) - jnp.maximum(b1y1, b2y1), 0.0)
            inter = iw * ih
            union = pw * ph + 1e-16 + tw * th - inter
            iou = inter / union
            cw = jnp.maximum(b1x2, b2x2) - jnp.minimum(b1x1, b2x1)
            ch = jnp.maximum(b1y2, b2y2) - jnp.minimum(b1y1, b2y1)
            c2 = cw * cw + ch * ch + 1e-16
            rho2 = ((b2x1 + b2x2 - b1x1 - b1x2) ** 2
                    + (b2y1 + b2y2 - b1y1 - b1y2) ** 2) * 0.25
            v = (4.0 / (math.pi ** 2)) * (_atan_pos(tw / th) - _atan_pos(pw / ph)) ** 2
            alpha = v / (1.0 - iou + v + 1e-16)
            ciou = iou - (rho2 / c2 + v * alpha)
            val_ref[s, ci] = 0.5 + 0.5 * jnp.maximum(ciou, 0.0)
            eg = ((s * 15 + ci) * 32 * 128
                  + lax.broadcasted_iota(jnp.int32, (32, 128), 0) * 128
                  + lax.broadcasted_iota(jnp.int32, (32, 128), 1))
            egl = jnp.where(eg < 3 * NE // 2, eg, eg - 3 * NE // 2)
            im = im_ref[s, ci] + MAPOFF[s]
            sidx_ref[s, ci] = jnp.where(mk > 0.5, im, TOT + egl)
        sums_ref[s, 0] = jnp.sum(acc_sl1, axis=0)
        sums_ref[s, 1] = jnp.sum(acc_bce, axis=0)
        sums_ref[s, 2] = jnp.sum(acc_cnt, axis=0)
        for r in range(3, 8):
            sums_ref[s, r] = jnp.zeros((128,), jnp.float32)


def _k_entry(g, ft, im):
    return pl.pallas_call(
        _entry_body,
        out_shape=(jax.ShapeDtypeStruct((3, 15, 32, 128), jnp.float32),
                   jax.ShapeDtypeStruct((3, 15, 32, 128), jnp.int32),
                   jax.ShapeDtypeStruct((3, 8, 128), jnp.float32)),
        interpret=_INTERPRET,
    )(g, ft, im)


def _fin_body(tobj_ref, d0_ref, d1_ref, d2_ref, o0_ref, o1_ref, o2_ref,
              sums_ref, out_ref):
    r0 = 1536
    r1 = 1920
    tob = jnp.maximum(tobj_ref[0], tobj_ref[1])
    st = [jnp.sum(tob[0:r0] * d0_ref[...]),
          jnp.sum(tob[r0:r1] * d1_ref[...]),
          jnp.sum(tob[r1:2016] * d2_ref[...])]
    osum = [jnp.sum(o0_ref[...]), jnp.sum(o1_ref[...]), jnp.sum(o2_ref[...])]
    lobj = jnp.float32(0.0)
    lbox = jnp.float32(0.0)
    lcls = jnp.float32(0.0)
    for s in range(3):
        lobj = lobj + BAL[s] * (-osum[s] + st[s]) / float(NPOS[s])
        sl1 = jnp.sum(sums_ref[s, 0])
        bce = jnp.sum(sums_ref[s, 1])
        cnt = jnp.sum(sums_ref[s, 2])
        den = jnp.maximum(cnt, 1.0)
        lbox = lbox + sl1 / (den * 4.0)
        lcls = lcls + bce / (den * float(NC))
    lbox = lbox * 0.05
    lobj = lobj * 1.4
    lcls = lcls * 0.5
    loss = (lbox + lobj + lcls) * float(B)
    out_ref[0] = jnp.full((128,), loss, jnp.float32)
    out_ref[1] = jnp.full((128,), lbox, jnp.float32)
    out_ref[2] = jnp.full((128,), lobj, jnp.float32)
    out_ref[3] = jnp.full((128,), lcls, jnp.float32)


def _k_fin(tobj2d, d0, d1, d2, o0, o1, o2, sums):
    return pl.pallas_call(
        _fin_body,
        out_shape=jax.ShapeDtypeStruct((4, 128), jnp.float32),
        interpret=_INTERPRET,
    )(tobj2d, d0, d1, d2, o0, o1, o2, sums)


_EPW = (3 * NE) // 32
_SPM = TOT + (3 * NE) // 2
_MPT = TOT // 16


def _scatter_body(sidx_hbm, val_hbm, zer_hbm, out_hbm, idx_v, val_v, spm, sem):
    cid = lax.axis_index("c")
    sid = lax.axis_index("s")
    w = cid * 16 + sid
    pltpu.sync_copy(zer_hbm, spm.at[pl.ds(sid * _MPT, _MPT)])
    plsc.subcore_barrier()
    pltpu.sync_copy(sidx_hbm.at[w], idx_v)
    pltpu.sync_copy(val_hbm.at[w], val_v)
    pltpu.async_copy(val_v, spm.at[idx_v], sem).wait()
    plsc.subcore_barrier()
    pltpu.sync_copy(spm.at[pl.ds(sid * _MPT, _MPT)],
                    out_hbm.at[cid, pl.ds(sid * _MPT, _MPT)])


def _sc_scatter(sidx, val):
    f = pl.kernel(
        _scatter_body,
        out_type=jax.ShapeDtypeStruct((2, TOT), jnp.float32),
        mesh=plsc.VectorSubcoreMesh(core_axis_name="c", subcore_axis_name="s"),
        scratch_types=[
            pltpu.VMEM((_EPW,), jnp.int32),
            pltpu.VMEM((_EPW,), jnp.float32),
            pltpu.VMEM_SHARED((_SPM,), jnp.float32),
            pltpu.SemaphoreType.DMA,
        ],
    )
    zer = jnp.zeros((_MPT,), jnp.float32)
    return f(sidx.reshape(32, _EPW), val.reshape(32, _EPW), zer)


_GPW = NE // 32


def _gather_body(p0_hbm, p1_hbm, p2_hbm, s0_hbm, s1_hbm, s2_hbm, idx_hbm,
                 out_hbm, idx_v, g_v, sem):
    cid = lax.axis_index("c")
    sid = lax.axis_index("s")
    w = cid * 16 + sid
    base = w * _GPW
    tabs = [p0_hbm, p1_hbm, p2_hbm, s0_hbm, s1_hbm, s2_hbm]
    hs = []
    for k in range(18):
        hs.append(pltpu.async_copy(idx_hbm.at[k, pl.ds(base, _GPW)],
                                   idx_v.at[pl.ds(k * _GPW, _GPW)], sem))
    for h in hs:
        h.wait()
    hs = []
    for k in range(18):
        s, slot = k // 6, k % 6
        tab = tabs[s] if slot < 5 else tabs[3 + s]
        hs.append(pltpu.async_copy(
            tab.at[idx_v.at[pl.ds(k * _GPW, _GPW)]],
            g_v.at[pl.ds(k * _GPW, _GPW)], sem))
    for h in hs:
        h.wait()
    hs = []
    for k in range(18):
        hs.append(pltpu.async_copy(g_v.at[pl.ds(k * _GPW, _GPW)],
                                   out_hbm.at[k, pl.ds(base, _GPW)], sem))
    for h in hs:
        h.wait()


def _sc_gather(pf, sf, idx):
    f = pl.kernel(
        _gather_body,
        out_type=jax.ShapeDtypeStruct((18, NE), jnp.float32),
        mesh=plsc.VectorSubcoreMesh(core_axis_name="c", subcore_axis_name="s"),
        scratch_types=[
            pltpu.VMEM((18 * _GPW,), jnp.int32),
            pltpu.VMEM((18 * _GPW,), jnp.float32),
            pltpu.SemaphoreType.DMA,
        ],
    )
    return f(pf[0], pf[1], pf[2], sf[0], sf[1], sf[2], idx.reshape(18, NE))


def kernel(pred0, pred1, pred2, targets):
    preds = [pred0, pred1, pred2]
    pf = [p.reshape(-1) for p in preds]
    tt = targets.T.reshape(6, 32, 128)
    idx, ft = _k_prep(tt)
    dense = [_k_dense(pf[s].reshape(B, 255, HWS[s] // 128, 128), s)
             for s in range(3)]
    sf = [dense[s][0].reshape(-1) for s in range(3)]
    g = _sc_gather(pf, sf, idx).reshape(3, 6, 15, 32, 128)
    val, sidx, sums = _k_entry(g, ft, idx[:, 5])
    tobj2d = _sc_scatter(sidx, val).reshape(2, 2016, 128)
    out = _k_fin(tobj2d,
                 dense[0][1].reshape(1536, 128),
                 dense[1][1].reshape(384, 128),
                 dense[2][1].reshape(96, 128),
                 dense[0][2].reshape(48, 128),
                 dense[1][2].reshape(48, 128),
                 dense[2][2].reshape(48, 128),
                 sums)
    return out[0, :1], out[1:4, 0]


# speedup vs baseline: 2.7163x; 1.0380x over previous
import functools
import math

import jax
import jax.numpy as jnp
from jax import lax
from jax.experimental import pallas as pl
from jax.experimental.pallas import tpu as pltpu
from jax.experimental.pallas import tpu_sc as plsc

_INTERPRET = False

B = 16
NA = 3
NC = 80
NT = 4096
HWS = [4096, 1024, 256]
WS = [64, 32, 16]
NPOS = [196608, 49152, 12288]
MAPOFF = [0, 196608, 245760]
TOT = 258048
TOBJ_PAD = TOT + 2 * 3 * 61440
DUMMY = 258048
NE = 61440
BAL = [4.0, 1.0, 0.4]
ANCHORS_RAW = [[(10.0, 13.0), (16.0, 30.0), (33.0, 23.0)],
               [(30.0, 61.0), (62.0, 45.0), (59.0, 119.0)],
               [(116.0, 90.0), (156.0, 198.0), (373.0, 326.0)]]
STRIDES = [8.0, 16.0, 32.0]
ANCH = [[(a / s, b / s) for (a, b) in ANCHORS_RAW[i]] for i, s in enumerate(STRIDES)]
OFFS = [(0.0, 0.0), (0.5, 0.0), (0.0, 0.5), (-0.5, 0.0), (0.0, -0.5)]


def _prep_body(t_ref, idx_ref, f_ref):
    img = t_ref[0]
    cls = t_ref[1]
    x = t_ref[2]
    y = t_ref[3]
    w = t_ref[4]
    h = t_ref[5]
    b = img.astype(jnp.int32)
    tc = cls.astype(jnp.int32)
    for s in range(3):
        W = float(WS[s])
        HW = HWS[s]
        gx = x * W
        gy = y * W
        gw = w * W
        gh = h * W
        fx = gx - jnp.floor(gx)
        fy = gy - jnp.floor(gy)
        jj = (fx < 0.5) & (gx > 1.0)
        kk = (fy < 0.5) & (gy > 1.0)
        gxi = W - gx
        gyi = W - gy
        fxi = gxi - jnp.floor(gxi)
        fyi = gyi - jnp.floor(gyi)
        ll = (fxi < 0.5) & (gxi > 1.0)
        mm = (fyi < 0.5) & (gyi > 1.0)
        gates = [None, jj, kk, ll, mm]
        m0 = []
        for a in range(NA):
            aw, ah = ANCH[s][a]
            rw = gw * (1.0 / aw)
            rh = gh * (1.0 / ah)
            mw = jnp.maximum(rw, 1.0 / rw)
            mh = jnp.maximum(rh, 1.0 / rh)
            m0.append(jnp.maximum(mw, mh) < 4.0)
        for o in range(5):
            ox, oy = OFFS[o]
            gi = (gx - ox).astype(jnp.int32)
            gj = (gy - oy).astype(jnp.int32)
            gi = jnp.clip(gi, 0, WS[s] - 1)
            gj = jnp.clip(gj, 0, WS[s] - 1)
            tx = gx - gi.astype(jnp.float32)
            ty = gy - gj.astype(jnp.float32)
            pos = gj * WS[s] + gi
            for a in range(NA):
                ci = o * NA + a
                base = (b * 255 + 85 * a) * HW + pos
                for c in range(4):
                    idx_ref[s, c, ci] = base + c * HW
                idx_ref[s, 4, ci] = base + (5 + tc) * HW
                idx_ref[s, 5, ci] = (b * NA + a) * HW + pos
                if gates[o] is None:
                    mk = m0[a]
                else:
                    mk = gates[o] & m0[a]
                f_ref[s, 0, ci] = tx
                f_ref[s, 1, ci] = ty
                f_ref[s, 2, ci] = gw
                f_ref[s, 3, ci] = gh
                f_ref[s, 4, ci] = mk.astype(jnp.float32)


def _k_prep(tt):
    return pl.pallas_call(
        _prep_body,
        out_shape=(jax.ShapeDtypeStruct((3, 6, 15, 32, 128), jnp.int32),
                   jax.ShapeDtypeStruct((3, 5, 15, 32, 128), jnp.float32)),
        interpret=_INTERPRET,
    )(tt)


def _dense_body(x_ref, s_ref, d_ref, os_ref, *, sub):
    def ch(c):
        return x_ref[0, c]

    rows = []
    for g in range(10):
        pr = 1.0 - ch(5 + 8 * g)
        for k in range(1, 8):
            pr = pr * (1.0 - ch(5 + 8 * g + k))
        rows.append(jnp.log(pr))
    acc = rows[0]
    for r in rows[1:]:
        acc = acc + r
    s_ref[0] = acc
    po = ch(4)
    lo1 = jnp.log(1.0 - po)
    lo0 = jnp.log(po)
    d_ref[0] = lo1 - lo0
    os_ref[0, 0] = jnp.sum(lo1, axis=0)


def _k_dense(p, s):
    hw = HWS[s]
    sub = hw // 128
    grid = (B * NA,)
    return pl.pallas_call(
        functools.partial(_dense_body, sub=sub),
        grid=grid,
        in_specs=[pl.BlockSpec((1, 85, sub, 128),
                               lambda i: (i // 3, i % 3, 0, 0))],
        out_specs=(pl.BlockSpec((1, sub, 128), lambda i: (i, 0, 0)),
                   pl.BlockSpec((1, sub, 128), lambda i: (i, 0, 0)),
                   pl.BlockSpec((1, 1, 128), lambda i: (i, 0, 0))),
        out_shape=(jax.ShapeDtypeStruct((B * NA, sub, 128), jnp.float32),
                   jax.ShapeDtypeStruct((B * NA, sub, 128), jnp.float32),
                   jax.ShapeDtypeStruct((B * NA, 1, 128), jnp.float32)),
        interpret=_INTERPRET,
    )(p)


def _atan_pos(z):
    inv = z > 1.0
    zz = jnp.where(inv, 1.0 / z, z)
    x2 = zz * zz
    p = -0.0117212
    p = p * x2 + 0.0529126
    p = p * x2 - 0.1169414
    p = p * x2 + 0.1939339
    p = p * x2 - 0.3326221
    p = p * x2 + 0.9999791
    at = p * zz
    return jnp.where(inv, (math.pi / 2.0) - at, at)


def _entry_body(g_ref, f_ref, im_ref, val_ref, sidx_ref, sums_ref):
    for s in range(3):
        acc_sl1 = jnp.zeros((32, 128), jnp.float32)
        acc_bce = jnp.zeros((32, 128), jnp.float32)
        acc_cnt = jnp.zeros((32, 128), jnp.float32)
        for ci in range(15):
            a = ci % NA
            aw, ah = ANCH[s][a]
            p0 = g_ref[s, 0, ci]
            p1 = g_ref[s, 1, ci]
            p2 = g_ref[s, 2, ci]
            p3 = g_ref[s, 3, ci]
            pct = g_ref[s, 4, ci]
            sv = g_ref[s, 5, ci]
            tx = f_ref[s, 0, ci]
            ty = f_ref[s, 1, ci]
            tw = f_ref[s, 2, ci]
            th = f_ref[s, 3, ci]
            mk = f_ref[s, 4, ci]
            px = p0 * 2.0 - 0.5
            py = p1 * 2.0 - 0.5
            pw = (p2 * 2.0) ** 2 * aw
            ph = (p3 * 2.0) ** 2 * ah
            sl1 = jnp.zeros((32, 128), jnp.float32)
            for pv, tv in ((px, tx), (py, ty), (pw, tw), (ph, th)):
                d = jnp.abs(pv - tv)
                sl1 = sl1 + jnp.where(d < 1.0, 0.5 * d * d, d - 0.5)
            acc_sl1 = acc_sl1 + mk * sl1
            bce = -jnp.log(pct) + jnp.log(1.0 - pct) - sv
            acc_bce = acc_bce + mk * bce
            acc_cnt = acc_cnt + mk
            b1x1 = px - pw * 0.5
            b1x2 = px + pw * 0.5
            b1y1 = py - ph * 0.5
            b1y2 = py + ph * 0.5
            b2x1 = tx - tw * 0.5
            b2x2 = tx + tw * 0.5
            b2y1 = ty - th * 0.5
            b2y2 = ty + th * 0.5
            iw = jnp.maximum(jnp.minimum(b1x2, b2x2) - jnp.maximum(b1x1, b2x1), 0.0)
            ih = jnp.maximum(jnp.minimum(b1y2, b2y2) - jnp.maximum(b1y1, b2y1), 0.0)
            inter = iw * ih
            union = pw * ph + 1e-16 + tw * th - inter
            iou = inter / union
            cw = jnp.maximum(b1x2, b2x2) - jnp.minimum(b1x1, b2x1)
            ch = jnp.maximum(b1y2, b2y2) - jnp.minimum(b1y1, b2y1)
            c2 = cw * cw + ch * ch + 1e-16
            rho2 = ((b2x1 + b2x2 - b1x1 - b1x2) ** 2
                    + (b2y1 + b2y2 - b1y1 - b1y2) ** 2) * 0.25
            v = (4.0 / (math.pi ** 2)) * (_atan_pos(tw / th) - _atan_pos(pw / ph)) ** 2
            alpha = v / (1.0 - iou + v + 1e-16)
            ciou = iou - (rho2 / c2 + v * alpha)
            val_ref[s, ci] = 0.5 + 0.5 * jnp.maximum(ciou, 0.0)
            eg = ((s * 15 + ci) * 32 * 128
                  + lax.broadcasted_iota(jnp.int32, (32, 128), 0) * 128
                  + lax.broadcasted_iota(jnp.int32, (32, 128), 1))
            egl = jnp.where(eg < 3 * NE // 2, eg, eg - 3 * NE // 2)
            im = im_ref[s, ci] + MAPOFF[s]
            sidx_ref[s, ci] = jnp.where(mk > 0.5, im, TOT + egl)
        sums_ref[s, 0] = jnp.sum(acc_sl1, axis=0)
        sums_ref[s, 1] = jnp.sum(acc_bce, axis=0)
        sums_ref[s, 2] = jnp.sum(acc_cnt, axis=0)
        for r in range(3, 8):
            sums_ref[s, r] = jnp.zeros((128,), jnp.float32)


def _k_entry(g, ft, im):
    return pl.pallas_call(
        _entry_body,
        out_shape=(jax.ShapeDtypeStruct((3, 15, 32, 128), jnp.float32),
                   jax.ShapeDtypeStruct((3, 15, 32, 128), jnp.int32),
                   jax.ShapeDtypeStruct((3, 8, 128), jnp.float32)),
        interpret=_INTERPRET,
    )(g, ft, im)


def _fin_body(tobj_ref, d0_ref, d1_ref, d2_ref, o0_ref, o1_ref, o2_ref,
              sums_ref, out_ref):
    r0 = 1536
    r1 = 1920
    tob = jnp.maximum(tobj_ref[0], tobj_ref[1])
    st = [jnp.sum(tob[0:r0] * d0_ref[...]),
          jnp.sum(tob[r0:r1] * d1_ref[...]),
          jnp.sum(tob[r1:2016] * d2_ref[...])]
    osum = [jnp.sum(o0_ref[...]), jnp.sum(o1_ref[...]), jnp.sum(o2_ref[...])]
    lobj = jnp.float32(0.0)
    lbox = jnp.float32(0.0)
    lcls = jnp.float32(0.0)
    for s in range(3):
        lobj = lobj + BAL[s] * (-osum[s] + st[s]) / float(NPOS[s])
        sl1 = jnp.sum(sums_ref[s, 0])
        bce = jnp.sum(sums_ref[s, 1])
        cnt = jnp.sum(sums_ref[s, 2])
        den = jnp.maximum(cnt, 1.0)
        lbox = lbox + sl1 / (den * 4.0)
        lcls = lcls + bce / (den * float(NC))
    lbox = lbox * 0.05
    lobj = lobj * 1.4
    lcls = lcls * 0.5
    loss = (lbox + lobj + lcls) * float(B)
    out_ref[0] = jnp.full((128,), loss, jnp.float32)
    out_ref[1] = jnp.full((128,), lbox, jnp.float32)
    out_ref[2] = jnp.full((128,), lobj, jnp.float32)
    out_ref[3] = jnp.full((128,), lcls, jnp.float32)


def _k_fin(tobj2d, d0, d1, d2, o0, o1, o2, sums):
    return pl.pallas_call(
        _fin_body,
        out_shape=jax.ShapeDtypeStruct((4, 128), jnp.float32),
        interpret=_INTERPRET,
    )(tobj2d, d0, d1, d2, o0, o1, o2, sums)


_EPW = (3 * NE) // 32
_SPM = TOT + (3 * NE) // 2
_MPT = TOT // 16


def _scatter_body(sidx_hbm, val_hbm, zer_hbm, out_hbm, idx_v, val_v, spm, sem):
    cid = lax.axis_index("c")
    sid = lax.axis_index("s")
    w = cid * 16 + sid
    pltpu.sync_copy(zer_hbm, spm.at[pl.ds(sid * _MPT, _MPT)])
    plsc.subcore_barrier()
    pltpu.sync_copy(sidx_hbm.at[pl.ds(w * _EPW, _EPW)], idx_v)
    pltpu.sync_copy(val_hbm.at[pl.ds(w * _EPW, _EPW)], val_v)
    pltpu.async_copy(val_v, spm.at[idx_v], sem).wait()
    plsc.subcore_barrier()
    pltpu.sync_copy(spm.at[pl.ds(sid * _MPT, _MPT)],
                    out_hbm.at[pl.ds(cid * TOT + sid * _MPT, _MPT)])


def _sc_scatter(sidx, val):
    f = pl.kernel(
        _scatter_body,
        out_type=jax.ShapeDtypeStruct((2 * TOT,), jnp.float32),
        mesh=plsc.VectorSubcoreMesh(core_axis_name="c", subcore_axis_name="s"),
        scratch_types=[
            pltpu.VMEM((_EPW,), jnp.int32),
            pltpu.VMEM((_EPW,), jnp.float32),
            pltpu.VMEM_SHARED((_SPM,), jnp.float32),
            pltpu.SemaphoreType.DMA,
        ],
    )
    zer = jnp.zeros((_MPT,), jnp.float32)
    return f(sidx.reshape(-1), val.reshape(-1), zer)


_GPW = NE // 32


def _gather_body(p0_hbm, p1_hbm, p2_hbm, s0_hbm, s1_hbm, s2_hbm, idx_hbm,
                 out_hbm, idx_v, g_v, sem):
    cid = lax.axis_index("c")
    sid = lax.axis_index("s")
    w = cid * 16 + sid
    tabs = [p0_hbm, p1_hbm, p2_hbm, s0_hbm, s1_hbm, s2_hbm]
    hs = []
    for k in range(18):
        hs.append(pltpu.async_copy(idx_hbm.at[pl.ds(k * NE + w * _GPW, _GPW)],
                                   idx_v.at[pl.ds(k * _GPW, _GPW)], sem))
    for h in hs:
        h.wait()
    hs = []
    for k in range(18):
        s, slot = k // 6, k % 6
        tab = tabs[s] if slot < 5 else tabs[3 + s]
        hs.append(pltpu.async_copy(
            tab.at[idx_v.at[pl.ds(k * _GPW, _GPW)]],
            g_v.at[pl.ds(k * _GPW, _GPW)], sem))
    for h in hs:
        h.wait()
    hs = []
    for k in range(18):
        hs.append(pltpu.async_copy(g_v.at[pl.ds(k * _GPW, _GPW)],
                                   out_hbm.at[pl.ds(k * NE + w * _GPW, _GPW)],
                                   sem))
    for h in hs:
        h.wait()


def _sc_gather(pf, sf, idx):
    f = pl.kernel(
        _gather_body,
        out_type=jax.ShapeDtypeStruct((18 * NE,), jnp.float32),
        mesh=plsc.VectorSubcoreMesh(core_axis_name="c", subcore_axis_name="s"),
        scratch_types=[
            pltpu.VMEM((18 * _GPW,), jnp.int32),
            pltpu.VMEM((18 * _GPW,), jnp.float32),
            pltpu.SemaphoreType.DMA,
        ],
    )
    return f(pf[0], pf[1], pf[2], sf[0], sf[1], sf[2], idx.reshape(-1))


def kernel(pred0, pred1, pred2, targets):
    preds = [pred0, pred1, pred2]
    p4 = [preds[s].reshape(B, 255, HWS[s] // 128, 128) for s in range(3)]
    pf = [p.reshape(-1) for p in p4]
    tt = targets.T.reshape(6, 32, 128)
    idx, ft = _k_prep(tt)
    dense = [_k_dense(p4[s], s) for s in range(3)]
    sf = [dense[s][0].reshape(-1) for s in range(3)]
    g = _sc_gather(pf, sf, idx).reshape(3, 6, 15, 32, 128)
    val, sidx, sums = _k_entry(g, ft, idx[:, 5])
    tobj2d = _sc_scatter(sidx, val).reshape(2, 2016, 128)
    out = _k_fin(tobj2d,
                 dense[0][1].reshape(1536, 128),
                 dense[1][1].reshape(384, 128),
                 dense[2][1].reshape(96, 128),
                 dense[0][2].reshape(48, 128),
                 dense[1][2].reshape(48, 128),
                 dense[2][2].reshape(48, 128),
                 sums)
    return out[0, :1], out[1:4, 0]
